# Initial kernel scaffold; baseline (speedup 1.0000x reference)
import jax
import jax.numpy as jnp
from jax.experimental import pallas as pl


def kernel(x, tables):
    raise NotImplementedError("write your pallas kernel here")


# speedup vs baseline: 1.1813x; 1.1813x over previous
import jax
import jax.numpy as jnp
from jax import lax
from jax.experimental import pallas as pl
from jax.experimental.pallas import tpu as pltpu
from jax.experimental.pallas import tpu_sc as plsc

_N_FIELDS = 26
_VOCAB = 100000
_EMBED = 32
_BATCH = 16384

_NC = 2
_NS = 16
_NW = _NC * _NS
_BPW = _BATCH // _NW
_C = 128
_NCHUNK = _BPW // _C
_L = 16
_IDXN = _C * _N_FIELDS
_STREAM = 128
_NSTREAM = _IDXN // _STREAM


def _embed_body(x_hbm, tbl_hbm, pat_hbm, out_hbm, idx_v, pat_v, rows_v,
                out_v, sem):
    wid = lax.axis_index("s") * _NC + lax.axis_index("c")
    base = wid * _BPW

    pltpu.sync_copy(pat_hbm, pat_v)

    for g in range(_NCHUNK):
        cbase = base + g * _C
        pltpu.sync_copy(x_hbm.at[pl.ds(cbase * _N_FIELDS, _IDXN)], idx_v)

        def _add_off(j, carry):
            o = j * _L
            idx_v[pl.ds(o, _L)] = idx_v[pl.ds(o, _L)] + pat_v[pl.ds(o, _L)]
            return carry

        lax.fori_loop(0, _IDXN // _L, _add_off, 0)

        cps = []
        for s in range(_NSTREAM):
            cps.append(pltpu.async_copy(
                tbl_hbm.at[idx_v.at[pl.ds(s * _STREAM, _STREAM)]],
                rows_v.at[pl.ds(s * _STREAM, _STREAM)],
                sem))
        for cp in cps:
            cp.wait()

        def _reduce(b, carry):
            r0 = b * _N_FIELDS
            a0 = rows_v[r0, pl.ds(0, _L)]
            a1 = rows_v[r0, pl.ds(_L, _L)]
            for i in range(1, _N_FIELDS):
                a0 = a0 + rows_v[r0 + i, pl.ds(0, _L)]
                a1 = a1 + rows_v[r0 + i, pl.ds(_L, _L)]
            out_v[b, pl.ds(0, _L)] = a0
            out_v[b, pl.ds(_L, _L)] = a1
            return carry

        lax.fori_loop(0, _C, _reduce, 0)

        pltpu.sync_copy(out_v, out_hbm.at[pl.ds(cbase, _C)])


def kernel(x, tables):
    x_flat = x.reshape(-1).astype(jnp.int32)
    tbl = tables.reshape(_N_FIELDS * _VOCAB, _EMBED)
    pat = jnp.tile(jnp.arange(_N_FIELDS, dtype=jnp.int32) * _VOCAB, _C)

    run = pl.kernel(
        _embed_body,
        out_type=jax.ShapeDtypeStruct((_BATCH, _EMBED), jnp.float32),
        mesh=plsc.VectorSubcoreMesh(core_axis_name="c", subcore_axis_name="s",
                                    num_cores=_NC, num_subcores=_NS),
        scratch_types=[
            pltpu.VMEM((_IDXN,), jnp.int32),
            pltpu.VMEM((_IDXN,), jnp.int32),
            pltpu.VMEM((_IDXN, _EMBED), jnp.float32),
            pltpu.VMEM((_C, _EMBED), jnp.float32),
            pltpu.SemaphoreType.DMA,
        ],
        compiler_params=pltpu.CompilerParams(use_tc_tiling_on_sc=False),
    )
    return run(x_flat, tbl, pat)


# speedup vs baseline: 3.7767x; 3.1971x over previous
import jax
import jax.numpy as jnp
from jax import lax
from jax.experimental import pallas as pl
from jax.experimental.pallas import tpu as pltpu
from jax.experimental.pallas import tpu_sc as plsc

_N_FIELDS = 26
_VOCAB = 100000
_EMBED = 32
_BATCH = 16384

_NC = 2
_NS = 16
_L = 16
_HALF = _BATCH // 2


def _embed_body(xt_hbm, tbl_hbm, out_hbm, row_v, idx_v, out_v, sem_r, sem_x):
    e = lax.axis_index("s") * _NC + lax.axis_index("c")

    for i in range(_N_FIELDS):
        row_cp = pltpu.async_copy(tbl_hbm.at[i * _EMBED + e], row_v, sem_r)
        for h in range(2):
            pltpu.async_copy(
                xt_hbm.at[i, pl.ds(h * _HALF, _HALF)], idx_v, sem_x).wait()
            if h == 0:
                row_cp.wait()

            if i == 0:
                def _first(j, carry):
                    g = plsc.load_gather(row_v, [idx_v[pl.ds(j * _L, _L)]])
                    out_v[pl.ds(h * _HALF + j * _L, _L)] = g
                    return carry
                lax.fori_loop(0, _HALF // _L, _first, 0)
            else:
                def _accum(j, carry):
                    g = plsc.load_gather(row_v, [idx_v[pl.ds(j * _L, _L)]])
                    plsc.addupdate(out_v.at[pl.ds(h * _HALF + j * _L, _L)], g)
                    return carry
                lax.fori_loop(0, _HALF // _L, _accum, 0)

    pltpu.sync_copy(out_v, out_hbm.at[e])


def kernel(x, tables):
    xt = x.astype(jnp.int32).T
    tbl = tables.transpose(0, 2, 1).reshape(_N_FIELDS * _EMBED, _VOCAB)

    run = pl.kernel(
        _embed_body,
        out_type=jax.ShapeDtypeStruct((_EMBED, _BATCH), jnp.float32),
        mesh=plsc.VectorSubcoreMesh(core_axis_name="c", subcore_axis_name="s",
                                    num_cores=_NC, num_subcores=_NS),
        scratch_types=[
            pltpu.VMEM((_VOCAB,), jnp.float32),
            pltpu.VMEM((_HALF,), jnp.int32),
            pltpu.VMEM((_BATCH,), jnp.float32),
            pltpu.SemaphoreType.DMA,
            pltpu.SemaphoreType.DMA,
        ],
        compiler_params=pltpu.CompilerParams(needs_layout_passes=False),
    )
    return run(xt, tbl).T


# speedup vs baseline: 6.0246x; 1.5952x over previous
import jax
import jax.numpy as jnp
from jax import lax
from jax.experimental import pallas as pl
from jax.experimental.pallas import tpu as pltpu
from jax.experimental.pallas import tpu_sc as plsc

_N_FIELDS = 26
_VOCAB = 100000
_EMBED = 32
_BATCH = 16384

_NC = 2
_NS = 16
_L = 16
_HALF = _BATCH // 2


def _embed_body(xt_hbm, tbl_hbm, out_hbm, row_v, idx_v, out_v, sem_r, sem_x):
    e = lax.axis_index("s") * _NC + lax.axis_index("c")

    for i in range(_N_FIELDS):
        row_cp = pltpu.async_copy(tbl_hbm.at[i * _EMBED + e], row_v, sem_r)
        for h in range(2):
            pltpu.async_copy(
                xt_hbm.at[i, pl.ds(h * _HALF, _HALF)], idx_v, sem_x).wait()
            if h == 0:
                row_cp.wait()

            if i == 0:
                @plsc.parallel_loop(0, _HALF, _L, unroll=8)
                def _first(o):
                    g = plsc.load_gather(row_v, [idx_v[pl.ds(o, _L)]])
                    out_v[pl.ds(h * _HALF + o, _L)] = g
            else:
                @plsc.parallel_loop(0, _HALF, _L, unroll=8)
                def _accum(o):
                    g = plsc.load_gather(row_v, [idx_v[pl.ds(o, _L)]])
                    plsc.addupdate(out_v.at[pl.ds(h * _HALF + o, _L)], g)

    pltpu.sync_copy(out_v, out_hbm.at[e])


def kernel(x, tables):
    xt = x.astype(jnp.int32).T
    tbl = tables.transpose(0, 2, 1).reshape(_N_FIELDS * _EMBED, _VOCAB)

    run = pl.kernel(
        _embed_body,
        out_type=jax.ShapeDtypeStruct((_EMBED, _BATCH), jnp.float32),
        mesh=plsc.VectorSubcoreMesh(core_axis_name="c", subcore_axis_name="s",
                                    num_cores=_NC, num_subcores=_NS),
        scratch_types=[
            pltpu.VMEM((_VOCAB,), jnp.float32),
            pltpu.VMEM((_HALF,), jnp.int32),
            pltpu.VMEM((_BATCH,), jnp.float32),
            pltpu.SemaphoreType.DMA,
            pltpu.SemaphoreType.DMA,
        ],
        compiler_params=pltpu.CompilerParams(needs_layout_passes=False),
    )
    return run(xt, tbl).T
